# Initial kernel scaffold; baseline (speedup 1.0000x reference)
import jax
import jax.numpy as jnp
from jax.experimental import pallas as pl


def kernel(x, router_w, w1_w, w1_b, w2_w, w2_b, w3_w, w3_b):
    raise NotImplementedError("write your pallas kernel here")


# speedup vs baseline: 968.3005x; 968.3005x over previous
import functools

import jax
import jax.numpy as jnp
from jax import lax
from jax.experimental import pallas as pl
from jax.experimental.pallas import tpu as pltpu
from jax.experimental.pallas import tpu_sc as plsc

_E = 16
_K = 4
_D = 2048
_DI = 1024
_B = 2
_S = 2048
_SK = _S * _K
_NW = 32
_CPE = _SK // _E
_RPT = (_B * _SK) // _NW
_TPT = (_B * _S) // _NW
_CG = 32
_EG = 8
_PB = 512


def _router_body(x_ref, rw_ref, dest_ref, score_ref):
    pid = pl.program_id(0)
    xb = x_ref[0]
    rw = rw_ref[...]
    st = lax.dot_general(
        rw, xb, (((1,), (1,)), ((), ())),
        preferred_element_type=jnp.float32)
    mx = jnp.max(st, axis=0, keepdims=True)
    ex = jnp.exp(st - mx)
    prob = ex / jnp.sum(ex, axis=0, keepdims=True)

    eidx = lax.broadcasted_iota(jnp.int32, (_E, _S), 0)
    s = st
    onehots = []
    for _ in range(_K):
        m = jnp.max(s, axis=0, keepdims=True)
        first = jnp.min(jnp.where(s == m, eidx, _E), axis=0, keepdims=True)
        oh = eidx == first
        onehots.append(oh.astype(jnp.float32))
        s = jnp.where(oh, -jnp.inf, s)
    mask = onehots[0] + onehots[1] + onehots[2] + onehots[3]

    ri = lax.broadcasted_iota(jnp.int32, (_S, _S), 0)
    ci = lax.broadcasted_iota(jnp.int32, (_S, _S), 1)
    utri = (ri <= ci).astype(jnp.bfloat16)
    c_incl = lax.dot_general(
        mask.astype(jnp.bfloat16), utri, (((1,), (0,)), ((), ())),
        preferred_element_type=jnp.float32)
    c_excl = c_incl - mask
    counts = c_incl[:, _S - 1:_S]
    er = lax.broadcasted_iota(jnp.int32, (_E, _E), 0)
    ec = lax.broadcasted_iota(jnp.int32, (_E, _E), 1)
    slt = (ec < er).astype(jnp.float32)
    offs = lax.dot_general(
        slt, counts, (((1,), (0,)), ((), ())),
        preferred_element_type=jnp.float32,
        precision=lax.Precision.HIGHEST)
    base = c_excl + offs

    dest_rows = [jnp.sum(oh * base, axis=0, keepdims=True) for oh in onehots]
    score_rows = [jnp.sum(oh * prob, axis=0, keepdims=True) for oh in onehots]
    dcat = jnp.concatenate(dest_rows, axis=0)
    dest_ref[0] = dcat.astype(jnp.int32) + pid * _SK
    score_ref[0] = jnp.concatenate(score_rows, axis=0)


def _router_call(x, router_w):
    return pl.pallas_call(
        _router_body,
        grid=(_B,),
        in_specs=[
            pl.BlockSpec((1, _S, _D), lambda b: (b, 0, 0)),
            pl.BlockSpec((_E, _D), lambda b: (0, 0)),
        ],
        out_specs=[
            pl.BlockSpec((1, _K, _S), lambda b: (b, 0, 0)),
            pl.BlockSpec((1, _K, _S), lambda b: (b, 0, 0)),
        ],
        out_shape=[
            jax.ShapeDtypeStruct((_B, _K, _S), jnp.int32),
            jax.ShapeDtypeStruct((_B, _K, _S), jnp.float32),
        ],
    )(x, router_w)


def _invert_body(dest_ref, score_ref, inv_ref):
    b = pl.program_id(0)
    blk = pl.program_id(1)
    pcol = (lax.broadcasted_iota(jnp.int32, (_PB, 1), 0)
            + (blk * _PB + b * _SK)).astype(jnp.float32)
    cmps = [(dest_ref[0][k:k + 1, :].astype(jnp.float32) == pcol)
            .astype(jnp.float32) for k in range(_K)]
    cmp = jnp.concatenate(cmps, axis=1)
    tok_row = (lax.broadcasted_iota(jnp.int32, (1, _S), 1)
               + b * _S).astype(jnp.float32)
    vt = jnp.concatenate(
        [jnp.concatenate([tok_row] * _K, axis=1),
         jnp.concatenate([score_ref[0][k:k + 1, :] for k in range(_K)],
                         axis=1)], axis=0)
    inv_ref[0, 0] = lax.dot_general(
        cmp, vt, (((1,), (1,)), ((), ())),
        preferred_element_type=jnp.float32,
        precision=lax.Precision.HIGHEST)


def _invert_call(dest, score):
    return pl.pallas_call(
        _invert_body,
        grid=(_B, _SK // _PB),
        in_specs=[
            pl.BlockSpec((1, _K, _S), lambda b, p: (b, 0, 0)),
            pl.BlockSpec((1, _K, _S), lambda b, p: (b, 0, 0)),
        ],
        out_specs=pl.BlockSpec((1, 1, _PB, 2), lambda b, p: (b, p, 0, 0)),
        out_shape=jax.ShapeDtypeStruct((_B, _SK // _PB, _PB, 2), jnp.float32),
    )(dest, score)


def _sc_mesh():
    return plsc.VectorSubcoreMesh(core_axis_name="c", subcore_axis_name="s")


@functools.cache
def _build_gather():
    return functools.partial(
        pl.kernel,
        out_type=jax.ShapeDtypeStruct((_B * _SK, _D // 2), jnp.int32),
        mesh=_sc_mesh(),
        scratch_types=[
            pltpu.VMEM((_CG,), jnp.int32),
            pltpu.VMEM((_CG, _D // 2), jnp.int32),
            pltpu.SemaphoreType.DMA,
        ],
    )(_gather_body)


def _gather_body(x_hbm, tok_hbm, xs_out, idxv, rows, sem):
    wid = lax.axis_index("s") * 2 + lax.axis_index("c")
    rbase = wid * _RPT

    def chunk(c, carry):
        off = rbase + c * _CG
        pltpu.sync_copy(tok_hbm.at[pl.ds(off, _CG)], idxv)
        pltpu.async_copy(x_hbm.at[idxv], rows, sem).wait()
        pltpu.sync_copy(rows, xs_out.at[pl.ds(off, _CG)])
        return carry

    lax.fori_loop(0, _RPT // _CG, chunk, 0)


def _ffn_body(xs_ref, w1_ref, b1_ref, w3_ref, b3_ref, w2_ref, b2_ref,
              sc_ref, out_ref):
    xsb = xs_ref[...]
    h1 = lax.dot_general(
        xsb, w1_ref[0], (((1,), (1,)), ((), ())),
        preferred_element_type=jnp.float32) + b1_ref[0, 0]
    h3 = lax.dot_general(
        xsb, w3_ref[0], (((1,), (1,)), ((), ())),
        preferred_element_type=jnp.float32) + b3_ref[0, 0]
    h = (h1 * lax.logistic(h1) * h3).astype(jnp.bfloat16)
    out = lax.dot_general(
        h, w2_ref[0], (((1,), (1,)), ((), ())),
        preferred_element_type=jnp.float32) + b2_ref[0, 0]
    out_ref[...] = out * sc_ref[0, 0][:, None]


def _ffn_call(xs, w1, b1, w3, b3, w2, b2, score3):
    return pl.pallas_call(
        _ffn_body,
        grid=(_E, _B),
        in_specs=[
            pl.BlockSpec((_CPE, _D), lambda e, b: (b * _E + e, 0)),
            pl.BlockSpec((1, _DI, _D), lambda e, b: (e, 0, 0)),
            pl.BlockSpec((1, 1, _DI), lambda e, b: (e, 0, 0)),
            pl.BlockSpec((1, _DI, _D), lambda e, b: (e, 0, 0)),
            pl.BlockSpec((1, 1, _DI), lambda e, b: (e, 0, 0)),
            pl.BlockSpec((1, _D, _DI), lambda e, b: (e, 0, 0)),
            pl.BlockSpec((1, 1, _D), lambda e, b: (e, 0, 0)),
            pl.BlockSpec((1, 1, _CPE), lambda e, b: (b * _E + e, 0, 0)),
        ],
        out_specs=pl.BlockSpec((_CPE, _D), lambda e, b: (b * _E + e, 0)),
        out_shape=jax.ShapeDtypeStruct((_B * _SK, _D), jnp.float32),
    )(xs, w1, b1, w3, b3, w2, b2, score3)


@functools.cache
def _build_combine():
    return functools.partial(
        pl.kernel,
        out_type=jax.ShapeDtypeStruct((_B * _S, _D), jnp.float32),
        mesh=_sc_mesh(),
        scratch_types=[
            pltpu.VMEM((_K, _EG), jnp.int32),
            pltpu.VMEM((_K, _EG, _D), jnp.float32),
            pltpu.VMEM((_EG, _D), jnp.float32),
            pltpu.SemaphoreType.DMA,
        ],
    )(_combine_body)


def _combine_body(out_hbm, dest_hbm, res_out, idxv, rows, acc, sem):
    wid = lax.axis_index("s") * 2 + lax.axis_index("c")
    bb = wid // 16
    tl_base = (wid % 16) * _TPT

    def chunk(c, carry):
        tl = tl_base + c * _EG
        for k in range(_K):
            pltpu.sync_copy(dest_hbm.at[bb, k, pl.ds(tl, _EG)], idxv.at[k])
        cps = [pltpu.async_copy(out_hbm.at[idxv.at[k]], rows.at[k], sem)
               for k in range(_K)]
        for cp in cps:
            cp.wait()

        def col(ci, carry2):
            sl = pl.ds(ci * 16, 16)
            for j in range(_EG):
                acc[j, sl] = ((rows[0, j, sl] + rows[1, j, sl])
                              + (rows[2, j, sl] + rows[3, j, sl]))
            return carry2

        lax.fori_loop(0, _D // 16, col, 0)
        pltpu.sync_copy(acc, res_out.at[pl.ds(bb * _S + tl, _EG)])
        return carry

    lax.fori_loop(0, _TPT // _EG, chunk, 0)


def kernel(x, router_w, w1_w, w1_b, w2_w, w2_b, w3_w, w3_b):
    dest, score = _router_call(x, router_w)
    inv = _invert_call(dest, score)
    src_tok = inv[..., 0].astype(jnp.int32).reshape(_B * _SK)
    score_sorted = inv[..., 1]
    x_bf = x.astype(jnp.bfloat16).reshape(_B * _S, _D // 2, 2)
    x_i32 = lax.bitcast_convert_type(x_bf, jnp.int32)
    xs_i32 = _build_gather()(x_i32, src_tok)
    xs = lax.bitcast_convert_type(xs_i32, jnp.bfloat16).reshape(_B * _SK, _D)
    out = _ffn_call(
        xs,
        w1_w.astype(jnp.bfloat16), w1_b.reshape(_E, 1, _DI),
        w3_w.astype(jnp.bfloat16), w3_b.reshape(_E, 1, _DI),
        w2_w.astype(jnp.bfloat16), w2_b.reshape(_E, 1, _D),
        score_sorted.reshape(_B * _E, 1, _CPE))
    res = _build_combine()(out, dest)
    return res.reshape(_B, _S, _D)


# speedup vs baseline: 968.5268x; 1.0002x over previous
import functools

import jax
import jax.numpy as jnp
from jax import lax
from jax.experimental import pallas as pl
from jax.experimental.pallas import tpu as pltpu
from jax.experimental.pallas import tpu_sc as plsc

_E = 16
_K = 4
_D = 2048
_DI = 1024
_B = 2
_S = 2048
_SK = _S * _K
_NW = 32
_CPE = _SK // _E
_RPT = (_B * _SK) // _NW
_TPT = (_B * _S) // _NW
_CG = 32
_EG = 8
_PB = 512


def _router_body(x_ref, rw_ref, dest_ref, score_ref):
    pid = pl.program_id(0)
    xb = x_ref[0]
    rw = rw_ref[...]
    st = lax.dot_general(
        rw, xb, (((1,), (1,)), ((), ())),
        preferred_element_type=jnp.float32)
    mx = jnp.max(st, axis=0, keepdims=True)
    ex = jnp.exp(st - mx)
    prob = ex / jnp.sum(ex, axis=0, keepdims=True)

    eidx = lax.broadcasted_iota(jnp.int32, (_E, _S), 0)
    s = st
    onehots = []
    for _ in range(_K):
        m = jnp.max(s, axis=0, keepdims=True)
        first = jnp.min(jnp.where(s == m, eidx, _E), axis=0, keepdims=True)
        oh = eidx == first
        onehots.append(oh.astype(jnp.float32))
        s = jnp.where(oh, -jnp.inf, s)
    mask = onehots[0] + onehots[1] + onehots[2] + onehots[3]

    ri = lax.broadcasted_iota(jnp.int32, (_S, _S), 0)
    ci = lax.broadcasted_iota(jnp.int32, (_S, _S), 1)
    utri = (ri <= ci).astype(jnp.bfloat16)
    c_incl = lax.dot_general(
        mask.astype(jnp.bfloat16), utri, (((1,), (0,)), ((), ())),
        preferred_element_type=jnp.float32)
    c_excl = c_incl - mask
    counts = c_incl[:, _S - 1:_S]
    er = lax.broadcasted_iota(jnp.int32, (_E, _E), 0)
    ec = lax.broadcasted_iota(jnp.int32, (_E, _E), 1)
    slt = (ec < er).astype(jnp.float32)
    offs = lax.dot_general(
        slt, counts, (((1,), (0,)), ((), ())),
        preferred_element_type=jnp.float32,
        precision=lax.Precision.HIGHEST)
    base = c_excl + offs

    dest_rows = [jnp.sum(oh * base, axis=0, keepdims=True) for oh in onehots]
    score_rows = [jnp.sum(oh * prob, axis=0, keepdims=True) for oh in onehots]
    dcat = jnp.concatenate(dest_rows, axis=0)
    dest_ref[0] = dcat.astype(jnp.int32) + pid * _SK
    score_ref[0] = jnp.concatenate(score_rows, axis=0)


def _router_call(x, router_w):
    return pl.pallas_call(
        _router_body,
        grid=(_B,),
        in_specs=[
            pl.BlockSpec((1, _S, _D), lambda b: (b, 0, 0)),
            pl.BlockSpec((_E, _D), lambda b: (0, 0)),
        ],
        out_specs=[
            pl.BlockSpec((1, _K, _S), lambda b: (b, 0, 0)),
            pl.BlockSpec((1, _K, _S), lambda b: (b, 0, 0)),
        ],
        out_shape=[
            jax.ShapeDtypeStruct((_B, _K, _S), jnp.int32),
            jax.ShapeDtypeStruct((_B, _K, _S), jnp.float32),
        ],
    )(x, router_w)


def _invert_body(dest_ref, score_ref, inv_ref):
    b = pl.program_id(0)
    blk = pl.program_id(1)
    pcol = (lax.broadcasted_iota(jnp.int32, (_PB, 1), 0)
            + (blk * _PB + b * _SK)).astype(jnp.float32)
    cmps = [(dest_ref[0][k:k + 1, :].astype(jnp.float32) == pcol)
            .astype(jnp.float32) for k in range(_K)]
    cmp = jnp.concatenate(cmps, axis=1)
    tok_row = (lax.broadcasted_iota(jnp.int32, (1, _S), 1)
               + b * _S).astype(jnp.float32)
    vt = jnp.concatenate(
        [jnp.concatenate([tok_row] * _K, axis=1),
         jnp.concatenate([score_ref[0][k:k + 1, :] for k in range(_K)],
                         axis=1)], axis=0)
    inv_ref[0, 0] = lax.dot_general(
        cmp, vt, (((1,), (1,)), ((), ())),
        preferred_element_type=jnp.float32,
        precision=lax.Precision.HIGHEST)


def _invert_call(dest, score):
    return pl.pallas_call(
        _invert_body,
        grid=(_B, _SK // _PB),
        in_specs=[
            pl.BlockSpec((1, _K, _S), lambda b, p: (b, 0, 0)),
            pl.BlockSpec((1, _K, _S), lambda b, p: (b, 0, 0)),
        ],
        out_specs=pl.BlockSpec((1, 1, _PB, 2), lambda b, p: (b, p, 0, 0)),
        out_shape=jax.ShapeDtypeStruct((_B, _SK // _PB, _PB, 2), jnp.float32),
    )(dest, score)


def _sc_mesh():
    return plsc.VectorSubcoreMesh(core_axis_name="c", subcore_axis_name="s")


@functools.cache
def _build_gather():
    return functools.partial(
        pl.kernel,
        out_type=jax.ShapeDtypeStruct((_B * _SK, _D // 2), jnp.int32),
        mesh=_sc_mesh(),
        scratch_types=[
            pltpu.VMEM((_CG,), jnp.int32),
            pltpu.VMEM((_CG, _D // 2), jnp.int32),
            pltpu.SemaphoreType.DMA,
        ],
        compiler_params=pltpu.CompilerParams(use_tc_tiling_on_sc=True),
    )(_gather_body)


def _gather_body(x_hbm, tok_hbm, xs_out, idxv, rows, sem):
    wid = lax.axis_index("s") * 2 + lax.axis_index("c")
    rbase = wid * _RPT

    def chunk(c, carry):
        off = rbase + c * _CG
        pltpu.sync_copy(tok_hbm.at[pl.ds(off, _CG)], idxv)
        pltpu.async_copy(x_hbm.at[idxv], rows, sem).wait()
        pltpu.sync_copy(rows, xs_out.at[pl.ds(off, _CG)])
        return carry

    lax.fori_loop(0, _RPT // _CG, chunk, 0)


def _ffn_body(xs_ref, w1_ref, b1_ref, w3_ref, b3_ref, w2_ref, b2_ref,
              sc_ref, out_ref):
    xsb = xs_ref[...]
    h1 = lax.dot_general(
        xsb, w1_ref[0], (((1,), (1,)), ((), ())),
        preferred_element_type=jnp.float32) + b1_ref[0, 0]
    h3 = lax.dot_general(
        xsb, w3_ref[0], (((1,), (1,)), ((), ())),
        preferred_element_type=jnp.float32) + b3_ref[0, 0]
    h = (h1 * lax.logistic(h1) * h3).astype(jnp.bfloat16)
    out = lax.dot_general(
        h, w2_ref[0], (((1,), (1,)), ((), ())),
        preferred_element_type=jnp.float32) + b2_ref[0, 0]
    out_ref[...] = out * sc_ref[0, 0][:, None]


def _ffn_call(xs, w1, b1, w3, b3, w2, b2, score3):
    return pl.pallas_call(
        _ffn_body,
        grid=(_E, _B),
        in_specs=[
            pl.BlockSpec((_CPE, _D), lambda e, b: (b * _E + e, 0)),
            pl.BlockSpec((1, _DI, _D), lambda e, b: (e, 0, 0)),
            pl.BlockSpec((1, 1, _DI), lambda e, b: (e, 0, 0)),
            pl.BlockSpec((1, _DI, _D), lambda e, b: (e, 0, 0)),
            pl.BlockSpec((1, 1, _DI), lambda e, b: (e, 0, 0)),
            pl.BlockSpec((1, _D, _DI), lambda e, b: (e, 0, 0)),
            pl.BlockSpec((1, 1, _D), lambda e, b: (e, 0, 0)),
            pl.BlockSpec((1, 1, _CPE), lambda e, b: (b * _E + e, 0, 0)),
        ],
        out_specs=pl.BlockSpec((_CPE, _D), lambda e, b: (b * _E + e, 0)),
        out_shape=jax.ShapeDtypeStruct((_B * _SK, _D), jnp.float32),
    )(xs, w1, b1, w3, b3, w2, b2, score3)


@functools.cache
def _build_combine():
    return functools.partial(
        pl.kernel,
        out_type=jax.ShapeDtypeStruct((_B * _S, _D), jnp.float32),
        mesh=_sc_mesh(),
        scratch_types=[
            pltpu.VMEM((_K, _EG), jnp.int32),
            pltpu.VMEM((_K, _EG, _D), jnp.float32),
            pltpu.VMEM((_EG, _D), jnp.float32),
            pltpu.SemaphoreType.DMA,
        ],
        compiler_params=pltpu.CompilerParams(use_tc_tiling_on_sc=True),
    )(_combine_body)


def _combine_body(out_hbm, dest_hbm, res_out, idxv, rows, acc, sem):
    wid = lax.axis_index("s") * 2 + lax.axis_index("c")
    bb = wid // 16
    tl_base = (wid % 16) * _TPT

    def chunk(c, carry):
        tl = tl_base + c * _EG
        for k in range(_K):
            pltpu.sync_copy(dest_hbm.at[bb, k, pl.ds(tl, _EG)], idxv.at[k])
        cps = [pltpu.async_copy(out_hbm.at[idxv.at[k]], rows.at[k], sem)
               for k in range(_K)]
        for cp in cps:
            cp.wait()

        def col(ci, carry2):
            sl = pl.ds(ci * 16, 16)
            for j in range(_EG):
                acc[j, sl] = ((rows[0, j, sl] + rows[1, j, sl])
                              + (rows[2, j, sl] + rows[3, j, sl]))
            return carry2

        lax.fori_loop(0, _D // 16, col, 0)
        pltpu.sync_copy(acc, res_out.at[pl.ds(bb * _S + tl, _EG)])
        return carry

    lax.fori_loop(0, _TPT // _EG, chunk, 0)


def kernel(x, router_w, w1_w, w1_b, w2_w, w2_b, w3_w, w3_b):
    dest, score = _router_call(x, router_w)
    inv = _invert_call(dest, score)
    src_tok = inv[..., 0].astype(jnp.int32).reshape(_B * _SK)
    score_sorted = inv[..., 1]
    x_bf = x.astype(jnp.bfloat16).reshape(_B * _S, _D // 2, 2)
    x_i32 = lax.bitcast_convert_type(x_bf, jnp.int32)
    xs_i32 = _build_gather()(x_i32, src_tok)
    xs = lax.bitcast_convert_type(xs_i32, jnp.bfloat16).reshape(_B * _SK, _D)
    out = _ffn_call(
        xs,
        w1_w.astype(jnp.bfloat16), w1_b.reshape(_E, 1, _DI),
        w3_w.astype(jnp.bfloat16), w3_b.reshape(_E, 1, _DI),
        w2_w.astype(jnp.bfloat16), w2_b.reshape(_E, 1, _D),
        score_sorted.reshape(_B * _E, 1, _CPE))
    res = _build_combine()(out, dest)
    return res.reshape(_B, _S, _D)


# speedup vs baseline: 1502.1829x; 1.5510x over previous
import functools

import jax
import jax.numpy as jnp
from jax import lax
from jax.experimental import pallas as pl
from jax.experimental.pallas import tpu as pltpu
from jax.experimental.pallas import tpu_sc as plsc

_E = 16
_K = 4
_D = 2048
_DI = 1024
_B = 2
_S = 2048
_SK = _S * _K
_NW = 32
_CPE = _SK // _E
_RPT = (_B * _SK) // _NW
_TPT = (_B * _S) // _NW
_CG = 32
_EG = 8
_PB = 512


def _router_body(x_ref, rw_ref, dest_ref, score_ref):
    pid = pl.program_id(0)
    xb = x_ref[0]
    rw = rw_ref[...]
    st = lax.dot_general(
        rw, xb, (((1,), (1,)), ((), ())),
        preferred_element_type=jnp.float32)
    mx = jnp.max(st, axis=0, keepdims=True)
    ex = jnp.exp(st - mx)
    prob = ex / jnp.sum(ex, axis=0, keepdims=True)

    eidx = lax.broadcasted_iota(jnp.int32, (_E, _S), 0)
    s = st
    onehots = []
    for _ in range(_K):
        m = jnp.max(s, axis=0, keepdims=True)
        first = jnp.min(jnp.where(s == m, eidx, _E), axis=0, keepdims=True)
        oh = eidx == first
        onehots.append(oh.astype(jnp.float32))
        s = jnp.where(oh, -jnp.inf, s)
    mask = onehots[0] + onehots[1] + onehots[2] + onehots[3]

    ri = lax.broadcasted_iota(jnp.int32, (_S, _S), 0)
    ci = lax.broadcasted_iota(jnp.int32, (_S, _S), 1)
    utri = (ri <= ci).astype(jnp.bfloat16)
    c_incl = lax.dot_general(
        mask.astype(jnp.bfloat16), utri, (((1,), (0,)), ((), ())),
        preferred_element_type=jnp.float32)
    c_excl = c_incl - mask
    counts = c_incl[:, _S - 1:_S]
    er = lax.broadcasted_iota(jnp.int32, (_E, _E), 0)
    ec = lax.broadcasted_iota(jnp.int32, (_E, _E), 1)
    slt = (ec < er).astype(jnp.float32)
    offs = lax.dot_general(
        slt, counts, (((1,), (0,)), ((), ())),
        preferred_element_type=jnp.float32,
        precision=lax.Precision.HIGHEST)
    base = c_excl + offs

    dest_rows = [jnp.sum(oh * base, axis=0, keepdims=True) for oh in onehots]
    score_rows = [jnp.sum(oh * prob, axis=0, keepdims=True) for oh in onehots]
    dcat = jnp.concatenate(dest_rows, axis=0)
    dest_ref[0] = dcat.astype(jnp.int32) + pid * _SK
    score_ref[0] = jnp.concatenate(score_rows, axis=0)


def _router_call(x, router_w):
    return pl.pallas_call(
        _router_body,
        grid=(_B,),
        in_specs=[
            pl.BlockSpec((1, _S, _D), lambda b: (b, 0, 0)),
            pl.BlockSpec((_E, _D), lambda b: (0, 0)),
        ],
        out_specs=[
            pl.BlockSpec((1, _K, _S), lambda b: (b, 0, 0)),
            pl.BlockSpec((1, _K, _S), lambda b: (b, 0, 0)),
        ],
        out_shape=[
            jax.ShapeDtypeStruct((_B, _K, _S), jnp.int32),
            jax.ShapeDtypeStruct((_B, _K, _S), jnp.float32),
        ],
    )(x, router_w)


def _invert_body(dest_ref, score_ref, tok_ref, scr_ref):
    b = pl.program_id(0)
    blk = pl.program_id(1)
    pcol = (lax.broadcasted_iota(jnp.int32, (_PB, 1), 0)
            + (blk * _PB + b * _SK)).astype(jnp.float32)
    cmps = [(dest_ref[0][k:k + 1, :].astype(jnp.float32) == pcol)
            .astype(jnp.float32) for k in range(_K)]
    cmp = jnp.concatenate(cmps, axis=1)
    tok_row = (lax.broadcasted_iota(jnp.int32, (1, _S), 1)
               + b * _S).astype(jnp.float32)
    vt = jnp.concatenate(
        [jnp.concatenate([tok_row] * _K, axis=1),
         jnp.concatenate([score_ref[0][k:k + 1, :] for k in range(_K)],
                         axis=1)], axis=0)
    outb = lax.dot_general(
        cmp, vt, (((1,), (1,)), ((), ())),
        preferred_element_type=jnp.float32,
        precision=lax.Precision.HIGHEST)
    tok_ref[0, 0] = outb[:, 0:1].astype(jnp.int32)
    scr_ref[0, 0] = outb[:, 1:2]


def _invert_call(dest, score):
    return pl.pallas_call(
        _invert_body,
        grid=(_B, _SK // _PB),
        in_specs=[
            pl.BlockSpec((1, _K, _S), lambda b, p: (b, 0, 0)),
            pl.BlockSpec((1, _K, _S), lambda b, p: (b, 0, 0)),
        ],
        out_specs=[
            pl.BlockSpec((1, 1, _PB, 1), lambda b, p: (b, p, 0, 0)),
            pl.BlockSpec((1, 1, _PB, 1), lambda b, p: (b, p, 0, 0)),
        ],
        out_shape=[
            jax.ShapeDtypeStruct((_B, _SK // _PB, _PB, 1), jnp.int32),
            jax.ShapeDtypeStruct((_B, _SK // _PB, _PB, 1), jnp.float32),
        ],
    )(dest, score)


def _sc_mesh():
    return plsc.VectorSubcoreMesh(core_axis_name="c", subcore_axis_name="s")


@functools.cache
def _build_gather():
    return functools.partial(
        pl.kernel,
        out_type=jax.ShapeDtypeStruct((_B * _SK, _D), jnp.float32),
        mesh=_sc_mesh(),
        scratch_types=[
            pltpu.VMEM((_CG,), jnp.int32),
            pltpu.VMEM((_CG, _D), jnp.float32),
            pltpu.SemaphoreType.DMA,
        ],
        compiler_params=pltpu.CompilerParams(use_tc_tiling_on_sc=True),
    )(_gather_body)


def _gather_body(x_hbm, tok_hbm, xs_out, idxv, rows, sem):
    wid = lax.axis_index("s") * 2 + lax.axis_index("c")
    rbase = wid * _RPT

    def chunk(c, carry):
        off = rbase + c * _CG
        pltpu.sync_copy(tok_hbm.at[pl.ds(off, _CG)], idxv)
        pltpu.async_copy(x_hbm.at[idxv], rows, sem).wait()
        pltpu.sync_copy(rows, xs_out.at[pl.ds(off, _CG)])
        return carry

    lax.fori_loop(0, _RPT // _CG, chunk, 0)


def _ffn_body(xs_ref, w1_ref, b1_ref, w3_ref, b3_ref, w2_ref, b2_ref,
              sc_ref, out_ref):
    xsb = xs_ref[...].astype(jnp.bfloat16)
    h1 = lax.dot_general(
        xsb, w1_ref[0], (((1,), (1,)), ((), ())),
        preferred_element_type=jnp.float32) + b1_ref[0, 0]
    h3 = lax.dot_general(
        xsb, w3_ref[0], (((1,), (1,)), ((), ())),
        preferred_element_type=jnp.float32) + b3_ref[0, 0]
    h = (h1 * lax.logistic(h1) * h3).astype(jnp.bfloat16)
    out = lax.dot_general(
        h, w2_ref[0], (((1,), (1,)), ((), ())),
        preferred_element_type=jnp.float32) + b2_ref[0, 0]
    out_ref[...] = out * sc_ref[0, 0][:, None]


def _ffn_call(xs, w1, b1, w3, b3, w2, b2, score3):
    return pl.pallas_call(
        _ffn_body,
        grid=(_E, _B),
        in_specs=[
            pl.BlockSpec((_CPE, _D), lambda e, b: (b * _E + e, 0)),
            pl.BlockSpec((1, _DI, _D), lambda e, b: (e, 0, 0)),
            pl.BlockSpec((1, 1, _DI), lambda e, b: (e, 0, 0)),
            pl.BlockSpec((1, _DI, _D), lambda e, b: (e, 0, 0)),
            pl.BlockSpec((1, 1, _DI), lambda e, b: (e, 0, 0)),
            pl.BlockSpec((1, _D, _DI), lambda e, b: (e, 0, 0)),
            pl.BlockSpec((1, 1, _D), lambda e, b: (e, 0, 0)),
            pl.BlockSpec((1, 1, _CPE), lambda e, b: (b * _E + e, 0, 0)),
        ],
        out_specs=pl.BlockSpec((_CPE, _D), lambda e, b: (b * _E + e, 0)),
        out_shape=jax.ShapeDtypeStruct((_B * _SK, _D), jnp.float32),
    )(xs, w1, b1, w3, b3, w2, b2, score3)


@functools.cache
def _build_combine():
    return functools.partial(
        pl.kernel,
        out_type=jax.ShapeDtypeStruct((_B * _S, _D), jnp.float32),
        mesh=_sc_mesh(),
        scratch_types=[
            pltpu.VMEM((_K, _EG), jnp.int32),
            pltpu.VMEM((_K, _EG, _D), jnp.float32),
            pltpu.VMEM((_EG, _D), jnp.float32),
            pltpu.SemaphoreType.DMA,
        ],
        compiler_params=pltpu.CompilerParams(use_tc_tiling_on_sc=True),
    )(_combine_body)


def _combine_body(out_hbm, dest_hbm, res_out, idxv, rows, acc, sem):
    wid = lax.axis_index("s") * 2 + lax.axis_index("c")
    bb = wid // 16
    tl_base = (wid % 16) * _TPT

    def chunk(c, carry):
        tl = tl_base + c * _EG
        for k in range(_K):
            pltpu.sync_copy(dest_hbm.at[bb, k, pl.ds(tl, _EG)], idxv.at[k])
        cps = [pltpu.async_copy(out_hbm.at[idxv.at[k]], rows.at[k], sem)
               for k in range(_K)]
        for cp in cps:
            cp.wait()

        def col(ci, carry2):
            sl = pl.ds(ci * 16, 16)
            for j in range(_EG):
                acc[j, sl] = ((rows[0, j, sl] + rows[1, j, sl])
                              + (rows[2, j, sl] + rows[3, j, sl]))
            return carry2

        lax.fori_loop(0, _D // 16, col, 0)
        pltpu.sync_copy(acc, res_out.at[pl.ds(bb * _S + tl, _EG)])
        return carry

    lax.fori_loop(0, _TPT // _EG, chunk, 0)


def kernel(x, router_w, w1_w, w1_b, w2_w, w2_b, w3_w, w3_b):
    dest, score = _router_call(x, router_w)
    src_tok, score_sorted = _invert_call(dest, score)
    xs = _build_gather()(x.reshape(_B * _S, _D),
                         src_tok.reshape(_B * _SK))
    out = _ffn_call(
        xs,
        w1_w.astype(jnp.bfloat16), w1_b.reshape(_E, 1, _DI),
        w3_w.astype(jnp.bfloat16), w3_b.reshape(_E, 1, _DI),
        w2_w.astype(jnp.bfloat16), w2_b.reshape(_E, 1, _D),
        score_sorted.reshape(_B * _E, 1, _CPE))
    res = _build_combine()(out, dest)
    return res.reshape(_B, _S, _D)
